# Initial kernel scaffold; baseline (speedup 1.0000x reference)
import jax
import jax.numpy as jnp
from jax.experimental import pallas as pl


def kernel(z_e, Wd, bd, cb, Wu, bu):
    raise NotImplementedError("write your pallas kernel here")


# speedup vs baseline: 2.6507x; 2.6507x over previous
import jax
import jax.numpy as jnp
from jax.experimental import pallas as pl

_L = 8
_K = 1024
_CD = 16
_EPS = 1e-12


def _bf(x):
    return x.astype(jnp.bfloat16)


def _rvq_body(x_ref, wdt_ref, wut_ref, bu_ref, bdrow_ref, cb_ref, cbt_ref,
              zq_ref, codes_ref, loss_ref):
    f32 = jnp.float32
    res = x_ref[:, :]
    blk = res.shape[0]

    zq_cols = []
    code_cols = []
    loss_sum = jnp.zeros((), dtype=f32)
    for i in range(_L):
        lo = i * _CD
        hi = lo + _CD
        ze = jnp.dot(_bf(res), _bf(wdt_ref[:, lo:hi]),
                     preferred_element_type=f32) + bdrow_ref[:, lo:hi]
        zf2 = jnp.sum(ze * ze, axis=1, keepdims=True)
        nrm = jnp.sqrt(zf2)
        zf = ze / jnp.maximum(nrm, _EPS)
        cbt_i = cbt_ref[lo:hi, :]
        cn = jnp.sqrt(jnp.sum(cbt_i * cbt_i, axis=0, keepdims=True))
        cbnt = cbt_i / jnp.maximum(cn, _EPS)
        cbn2 = jnp.sum(cbnt * cbnt, axis=0, keepdims=True)
        zfn2 = jnp.sum(zf * zf, axis=1, keepdims=True)
        mm = jnp.dot(_bf(zf), _bf(cbnt), preferred_element_type=f32)
        d2 = (zfn2 + cbn2) - 2.0 * mm
        minv = jnp.min(d2, axis=1, keepdims=True)
        lane = jax.lax.broadcasted_iota(jnp.int32, (blk, _K), 1)
        codes = jnp.min(jnp.where(d2 == minv, lane, _K), axis=1, keepdims=True)
        oh = (lane == codes).astype(jnp.bfloat16)
        cb_i = cb_ref[i * _K:(i + 1) * _K, :]
        zq = jnp.dot(oh, _bf(cb_i), preferred_element_type=f32)
        zq_cols.append(ze + (zq - ze))
        code_cols.append(codes)
        d = ze - zq
        loss_sum = loss_sum + jnp.sum(d * d)
        up = jnp.dot(_bf(zq), _bf(wut_ref[lo:hi, :]),
                     preferred_element_type=f32)
        res = res - (up + bu_ref[i:i + 1, :])

    zq_ref[:, :] = jnp.concatenate(zq_cols, axis=1)
    codes_ref[:, :] = jnp.concatenate(code_cols, axis=1)
    loss_ref[0, :, :] = jnp.full((8, 128), loss_sum, dtype=f32)


def kernel(z_e, Wd, bd, cb, Wu, bu):
    Bc, Tc, Dc = z_e.shape
    n = Bc * Tc
    f32 = jnp.float32

    x = z_e.reshape(n, Dc)
    wdt = Wd.reshape(_L * _CD, Dc).T
    wut = Wu.transpose(0, 2, 1).reshape(_L * _CD, Dc)
    bdrow = bd.reshape(1, _L * _CD)
    cb2d = cb.reshape(_L * _K, _CD)
    cbt2d = cb.transpose(0, 2, 1).reshape(_L * _CD, _K)

    blk = 1024
    nb = n // blk
    grid = (nb,)

    zq_out, codes_out, loss_out = pl.pallas_call(
        _rvq_body,
        grid=grid,
        in_specs=[
            pl.BlockSpec((blk, Dc), lambda b: (b, 0)),
            pl.BlockSpec((Dc, _L * _CD), lambda b: (0, 0)),
            pl.BlockSpec((_L * _CD, Dc), lambda b: (0, 0)),
            pl.BlockSpec((_L, Dc), lambda b: (0, 0)),
            pl.BlockSpec((1, _L * _CD), lambda b: (0, 0)),
            pl.BlockSpec((_L * _K, _CD), lambda b: (0, 0)),
            pl.BlockSpec((_L * _CD, _K), lambda b: (0, 0)),
        ],
        out_specs=[
            pl.BlockSpec((blk, _L * _CD), lambda b: (b, 0)),
            pl.BlockSpec((blk, _L), lambda b: (b, 0)),
            pl.BlockSpec((1, 8, 128), lambda b: (b, 0, 0)),
        ],
        out_shape=[
            jax.ShapeDtypeStruct((n, _L * _CD), f32),
            jax.ShapeDtypeStruct((n, _L), jnp.int32),
            jax.ShapeDtypeStruct((nb, 8, 128), f32),
        ],
    )(x, wdt, wut, bu, bdrow, cb2d, cbt2d)

    z_q_concat = zq_out.reshape(Bc, Tc, _L * _CD)
    codes = codes_out.reshape(Bc, Tc, _L)
    total = jnp.sum(loss_out[:, 0, 0])
    commit = total / jnp.asarray(n * _CD, dtype=f32)
    cb_loss = total / jnp.asarray(n * _CD, dtype=f32)
    entropy_loss = jnp.zeros((), dtype=f32)
    return (z_q_concat, codes, commit, cb_loss, entropy_loss)
